# initial kernel scaffold (unmeasured)
import functools

import jax
import jax.numpy as jnp
from jax import lax
from jax.experimental import pallas as pl
from jax.experimental.pallas import tpu as pltpu

N_DEV = 4
M = 4096
N_OUT = 2048
CHUNK = M // N_DEV


def kernel(x, w_mat):
    def body(
        x_ref,
        w_ref,
        out_ref,
        acc_ref,
        rs_recv_ref,
        rs_send_sems,
        rs_recv_sems,
        ag_send_sems,
        ag_recv_sems,
    ):
        my = lax.axis_index("i")
        right = (my + 1) % N_DEV
        left = (my - 1) % N_DEV

        barrier_sem = pltpu.get_barrier_semaphore()
        for nbr in (left, right):
            pl.semaphore_signal(
                barrier_sem, inc=1,
                device_id=(nbr,), device_id_type=pl.DeviceIdType.MESH,
            )
        pl.semaphore_wait(barrier_sem, 2)

        def partial_chunk(c):
            rows = pl.ds(c * CHUNK, CHUNK)
            return jnp.dot(
                x_ref[rows, :], w_ref[...], preferred_element_type=jnp.float32
            )

        acc_ref[...] = partial_chunk(my)
        for h in range(N_DEV - 1):
            rdma = pltpu.make_async_remote_copy(
                src_ref=acc_ref,
                dst_ref=rs_recv_ref.at[h],
                send_sem=rs_send_sems.at[h],
                recv_sem=rs_recv_sems.at[h],
                device_id=(right,),
                device_id_type=pl.DeviceIdType.MESH,
            )
            rdma.start()
            p = partial_chunk((my - h - 1) % N_DEV)
            rdma.wait()
            acc_ref[...] = rs_recv_ref[h] + p

        own = (my + 1) % N_DEV
        out_ref[pl.ds(own * CHUNK, CHUNK), :] = acc_ref[...]

        for g in range(N_DEV - 1):
            src_c = (my + 1 - g) % N_DEV
            rows = pl.ds(src_c * CHUNK, CHUNK)
            rdma = pltpu.make_async_remote_copy(
                src_ref=out_ref.at[rows, :],
                dst_ref=out_ref.at[rows, :],
                send_sem=ag_send_sems.at[g],
                recv_sem=ag_recv_sems.at[g],
                device_id=(right,),
                device_id_type=pl.DeviceIdType.MESH,
            )
            rdma.start()
            rdma.wait()

        y = out_ref[...]
        amax = jnp.max(jnp.abs(y))
        scale = amax / 127.0
        q = jnp.clip(jnp.round(y / scale), -127.0, 127.0)
        out_ref[...] = q * scale

        @functools.partial(
            pl.run_scoped, second_barrier=pltpu.SemaphoreType.REGULAR
        )
        def _(second_barrier):
            for nbr in (left, right):
                pl.semaphore_signal(
                    second_barrier, inc=1,
                    device_id=(nbr,), device_id_type=pl.DeviceIdType.MESH,
                )
            pl.semaphore_wait(second_barrier, 2)

    return pl.pallas_call(
        body,
        out_shape=jax.ShapeDtypeStruct((M, N_OUT), jnp.float32),
        in_specs=[
            pl.BlockSpec(memory_space=pltpu.VMEM),
            pl.BlockSpec(memory_space=pltpu.VMEM),
        ],
        out_specs=pl.BlockSpec(memory_space=pltpu.VMEM),
        scratch_shapes=[
            pltpu.VMEM((CHUNK, N_OUT), jnp.float32),
            pltpu.VMEM((N_DEV - 1, CHUNK, N_OUT), jnp.float32),
            pltpu.SemaphoreType.DMA((N_DEV - 1,)),
            pltpu.SemaphoreType.DMA((N_DEV - 1,)),
            pltpu.SemaphoreType.DMA((N_DEV - 1,)),
            pltpu.SemaphoreType.DMA((N_DEV - 1,)),
        ],
        compiler_params=pltpu.CompilerParams(
            collective_id=0,
            vmem_limit_bytes=128 * 1024 * 1024,
        ),
    )(x, w_mat)


# baseline (device time: 408579 ns/iter reference)
import functools

import jax
import jax.numpy as jnp
from jax import lax
from jax.experimental import pallas as pl
from jax.experimental.pallas import tpu as pltpu

N_DEV = 4
M = 4096
K_SHARD = 1024
N_OUT = 2048
CHUNK = M // N_DEV


def kernel(x, w_mat):
    def body(
        x_hbm,
        w_ref,
        out_hbm,
        xs_ref,
        acc_ref,
        rs_recv_ref,
        q_ref,
        amax_ref,
        copy_sem,
        out_sem,
        rs_send_sems,
        rs_recv_sems,
        ax_send_sems,
        ax_recv_sems,
        ag_send_sems,
        ag_recv_sems,
    ):
        my = lax.axis_index("i")
        right = (my + 1) % N_DEV
        left = (my + N_DEV - 1) % N_DEV

        barrier_sem = pltpu.get_barrier_semaphore()
        for nbr in (left, right):
            pl.semaphore_signal(
                barrier_sem, inc=1,
                device_id=(nbr,), device_id_type=pl.DeviceIdType.MESH,
            )
        pl.semaphore_wait(barrier_sem, 2)

        def partial_chunk(c):
            cp = pltpu.make_async_copy(
                x_hbm.at[pl.ds(c * CHUNK, CHUNK), :], xs_ref, copy_sem
            )
            cp.start()
            cp.wait()
            return jnp.dot(
                xs_ref[...], w_ref[...], preferred_element_type=jnp.float32
            )

        acc_ref[...] = partial_chunk(my)
        for h in range(N_DEV - 1):
            rdma = pltpu.make_async_remote_copy(
                src_ref=acc_ref,
                dst_ref=rs_recv_ref.at[h],
                send_sem=rs_send_sems.at[h],
                recv_sem=rs_recv_sems.at[h],
                device_id=(right,),
                device_id_type=pl.DeviceIdType.MESH,
            )
            rdma.start()
            p = partial_chunk((my - h - 1) % N_DEV)
            rdma.wait()
            acc_ref[...] = rs_recv_ref[h] + p

        own = (my + 1) % N_DEV

        my_max = jnp.max(jnp.abs(acc_ref[...]))
        amax_ref[pl.ds(my, 1)] = jnp.full((1, 8, 128), my_max, jnp.float32)
        ax_rdmas = []
        for k in range(1, N_DEV):
            tgt = (my + k) % N_DEV
            rdma = pltpu.make_async_remote_copy(
                src_ref=amax_ref.at[pl.ds(my, 1)],
                dst_ref=amax_ref.at[pl.ds(my, 1)],
                send_sem=ax_send_sems.at[k - 1],
                recv_sem=ax_recv_sems.at[k - 1],
                device_id=(tgt,),
                device_id_type=pl.DeviceIdType.MESH,
            )
            rdma.start()
            ax_rdmas.append(rdma)
        for rdma in ax_rdmas:
            rdma.wait()
        amax = jnp.max(amax_ref[...])
        scale = amax / 127.0

        q = jnp.clip(jnp.round(acc_ref[...] / scale), -127.0, 127.0)
        q_ref[pl.ds(own, 1)] = q.astype(jnp.int8)[None]

        for g in range(N_DEV - 1):
            src_c = (my + 1 - g) % N_DEV
            rdma = pltpu.make_async_remote_copy(
                src_ref=q_ref.at[pl.ds(src_c, 1)],
                dst_ref=q_ref.at[pl.ds(src_c, 1)],
                send_sem=ag_send_sems.at[g],
                recv_sem=ag_recv_sems.at[g],
                device_id=(right,),
                device_id_type=pl.DeviceIdType.MESH,
            )
            rdma.start()
            rdma.wait()

        for c in range(N_DEV):
            acc_ref[...] = q_ref[c].astype(jnp.float32) * scale
            cp = pltpu.make_async_copy(
                acc_ref, out_hbm.at[pl.ds(c * CHUNK, CHUNK), :], out_sem
            )
            cp.start()
            cp.wait()

        @functools.partial(
            pl.run_scoped, second_barrier=pltpu.SemaphoreType.REGULAR
        )
        def _(second_barrier):
            for nbr in (left, right):
                pl.semaphore_signal(
                    second_barrier, inc=1,
                    device_id=(nbr,), device_id_type=pl.DeviceIdType.MESH,
                )
            pl.semaphore_wait(second_barrier, 2)

    return pl.pallas_call(
        body,
        out_shape=jax.ShapeDtypeStruct((M, N_OUT), jnp.float32),
        in_specs=[
            pl.BlockSpec(memory_space=pl.ANY),
            pl.BlockSpec(memory_space=pltpu.VMEM),
        ],
        out_specs=pl.BlockSpec(memory_space=pl.ANY),
        scratch_shapes=[
            pltpu.VMEM((CHUNK, K_SHARD), jnp.float32),
            pltpu.VMEM((CHUNK, N_OUT), jnp.float32),
            pltpu.VMEM((N_DEV - 1, CHUNK, N_OUT), jnp.float32),
            pltpu.VMEM((N_DEV, CHUNK, N_OUT), jnp.int8),
            pltpu.VMEM((N_DEV, 8, 128), jnp.float32),
            pltpu.SemaphoreType.DMA,
            pltpu.SemaphoreType.DMA,
            pltpu.SemaphoreType.DMA((N_DEV - 1,)),
            pltpu.SemaphoreType.DMA((N_DEV - 1,)),
            pltpu.SemaphoreType.DMA((N_DEV - 1,)),
            pltpu.SemaphoreType.DMA((N_DEV - 1,)),
            pltpu.SemaphoreType.DMA((N_DEV - 1,)),
            pltpu.SemaphoreType.DMA((N_DEV - 1,)),
        ],
        compiler_params=pltpu.CompilerParams(
            collective_id=0,
            vmem_limit_bytes=128 * 1024 * 1024,
        ),
    )(x, w_mat)


# device time: 228292 ns/iter; 1.7897x vs baseline; 1.7897x over previous
import functools

import jax
import jax.numpy as jnp
from jax import lax
from jax.experimental import pallas as pl
from jax.experimental.pallas import tpu as pltpu

N_DEV = 4
M = 4096
K_SHARD = 1024
N_OUT = 2048
HALF = N_OUT // 2
CHUNK = M // N_DEV


def kernel(x, w_mat):
    def body(
        x_hbm,
        w_ref,
        out_hbm,
        xs_ref,
        acc_r_ref,
        acc_l_ref,
        rsr_ref,
        rsl_ref,
        qr_ref,
        ql_ref,
        amax_ref,
        copy_sem,
        out_sem,
        rs_send_r,
        rs_recv_r,
        rs_send_l,
        rs_recv_l,
        ax_send,
        ax_recv,
        ag_send_r,
        ag_recv_r,
        ag_send_l,
        ag_recv_l,
    ):
        my = lax.axis_index("i")
        right = (my + 1) % N_DEV
        left = (my + N_DEV - 1) % N_DEV

        barrier_sem = pltpu.get_barrier_semaphore()
        for nbr in (left, right):
            pl.semaphore_signal(
                barrier_sem, inc=1,
                device_id=(nbr,), device_id_type=pl.DeviceIdType.MESH,
            )
        pl.semaphore_wait(barrier_sem, 2)

        def load_x(c):
            cp = pltpu.make_async_copy(
                x_hbm.at[pl.ds(c * CHUNK, CHUNK), :], xs_ref, copy_sem
            )
            cp.start()
            cp.wait()

        def dot_half(lo):
            return jnp.dot(
                xs_ref[...],
                w_ref[:, lo : lo + HALF],
                preferred_element_type=jnp.float32,
            )

        load_x(my)
        acc_r_ref[...] = dot_half(0)
        acc_l_ref[...] = dot_half(HALF)
        for h in range(N_DEV - 1):
            rdma_r = pltpu.make_async_remote_copy(
                src_ref=acc_r_ref,
                dst_ref=rsr_ref.at[h],
                send_sem=rs_send_r.at[h],
                recv_sem=rs_recv_r.at[h],
                device_id=(right,),
                device_id_type=pl.DeviceIdType.MESH,
            )
            rdma_l = pltpu.make_async_remote_copy(
                src_ref=acc_l_ref,
                dst_ref=rsl_ref.at[h],
                send_sem=rs_send_l.at[h],
                recv_sem=rs_recv_l.at[h],
                device_id=(left,),
                device_id_type=pl.DeviceIdType.MESH,
            )
            rdma_r.start()
            rdma_l.start()
            load_x((my - h - 1) % N_DEV)
            p_r = dot_half(0)
            if h != 1:
                load_x((my + h + 1) % N_DEV)
            p_l = dot_half(HALF)
            rdma_r.wait()
            acc_r_ref[...] = rsr_ref[h] + p_r
            rdma_l.wait()
            acc_l_ref[...] = rsl_ref[h] + p_l

        own_r = (my + 1) % N_DEV
        own_l = (my + N_DEV - 1) % N_DEV

        my_max = jnp.maximum(
            jnp.max(jnp.abs(acc_r_ref[...])), jnp.max(jnp.abs(acc_l_ref[...]))
        )
        amax_ref[pl.ds(my, 1)] = jnp.full((1, 8, 128), my_max, jnp.float32)
        ax_rdmas = []
        for k in range(1, N_DEV):
            rdma = pltpu.make_async_remote_copy(
                src_ref=amax_ref.at[pl.ds(my, 1)],
                dst_ref=amax_ref.at[pl.ds(my, 1)],
                send_sem=ax_send.at[k - 1],
                recv_sem=ax_recv.at[k - 1],
                device_id=((my + k) % N_DEV,),
                device_id_type=pl.DeviceIdType.MESH,
            )
            rdma.start()
            ax_rdmas.append(rdma)
        for rdma in ax_rdmas:
            rdma.wait()
        amax = jnp.max(amax_ref[...])
        scale = amax / 127.0

        def quantize(v):
            return jnp.clip(jnp.round(v / scale), -127.0, 127.0).astype(
                jnp.int8
            )

        qr_ref[pl.ds(own_r, 1)] = quantize(acc_r_ref[...])[None]
        ql_ref[pl.ds(own_l, 1)] = quantize(acc_l_ref[...])[None]

        def store_half(q_ref_, stage_ref, c, col_lo):
            stage_ref[...] = q_ref_[pl.ds(c, 1)][0].astype(jnp.float32) * scale
            cp = pltpu.make_async_copy(
                stage_ref,
                out_hbm.at[pl.ds(c * CHUNK, CHUNK), pl.ds(col_lo, HALF)],
                out_sem,
            )
            cp.start()
            cp.wait()

        for g in range(N_DEV - 1):
            rdma_r = pltpu.make_async_remote_copy(
                src_ref=qr_ref.at[pl.ds((my + 1 - g) % N_DEV, 1)],
                dst_ref=qr_ref.at[pl.ds((my + 1 - g) % N_DEV, 1)],
                send_sem=ag_send_r.at[g],
                recv_sem=ag_recv_r.at[g],
                device_id=(right,),
                device_id_type=pl.DeviceIdType.MESH,
            )
            rdma_l = pltpu.make_async_remote_copy(
                src_ref=ql_ref.at[pl.ds((my - 1 + g) % N_DEV, 1)],
                dst_ref=ql_ref.at[pl.ds((my - 1 + g) % N_DEV, 1)],
                send_sem=ag_send_l.at[g],
                recv_sem=ag_recv_l.at[g],
                device_id=(left,),
                device_id_type=pl.DeviceIdType.MESH,
            )
            rdma_r.start()
            rdma_l.start()
            store_half(qr_ref, acc_r_ref, (my + 1 - g) % N_DEV, 0)
            store_half(ql_ref, acc_l_ref, (my - 1 + g) % N_DEV, HALF)
            rdma_r.wait()
            rdma_l.wait()
        store_half(qr_ref, acc_r_ref, (my - 2) % N_DEV, 0)
        store_half(ql_ref, acc_l_ref, (my + 2) % N_DEV, HALF)

        @functools.partial(
            pl.run_scoped, second_barrier=pltpu.SemaphoreType.REGULAR
        )
        def _(second_barrier):
            for nbr in (left, right):
                pl.semaphore_signal(
                    second_barrier, inc=1,
                    device_id=(nbr,), device_id_type=pl.DeviceIdType.MESH,
                )
            pl.semaphore_wait(second_barrier, 2)

    return pl.pallas_call(
        body,
        out_shape=jax.ShapeDtypeStruct((M, N_OUT), jnp.float32),
        in_specs=[
            pl.BlockSpec(memory_space=pl.ANY),
            pl.BlockSpec(memory_space=pltpu.VMEM),
        ],
        out_specs=pl.BlockSpec(memory_space=pl.ANY),
        scratch_shapes=[
            pltpu.VMEM((CHUNK, K_SHARD), jnp.float32),
            pltpu.VMEM((CHUNK, HALF), jnp.float32),
            pltpu.VMEM((CHUNK, HALF), jnp.float32),
            pltpu.VMEM((N_DEV - 1, CHUNK, HALF), jnp.float32),
            pltpu.VMEM((N_DEV - 1, CHUNK, HALF), jnp.float32),
            pltpu.VMEM((N_DEV, CHUNK, HALF), jnp.int8),
            pltpu.VMEM((N_DEV, CHUNK, HALF), jnp.int8),
            pltpu.VMEM((N_DEV, 8, 128), jnp.float32),
            pltpu.SemaphoreType.DMA,
            pltpu.SemaphoreType.DMA,
            pltpu.SemaphoreType.DMA((N_DEV - 1,)),
            pltpu.SemaphoreType.DMA((N_DEV - 1,)),
            pltpu.SemaphoreType.DMA((N_DEV - 1,)),
            pltpu.SemaphoreType.DMA((N_DEV - 1,)),
            pltpu.SemaphoreType.DMA((N_DEV - 1,)),
            pltpu.SemaphoreType.DMA((N_DEV - 1,)),
            pltpu.SemaphoreType.DMA((N_DEV - 1,)),
            pltpu.SemaphoreType.DMA((N_DEV - 1,)),
            pltpu.SemaphoreType.DMA((N_DEV - 1,)),
            pltpu.SemaphoreType.DMA((N_DEV - 1,)),
        ],
        compiler_params=pltpu.CompilerParams(
            collective_id=0,
            vmem_limit_bytes=128 * 1024 * 1024,
        ),
    )(x, w_mat)


# device time: 159257 ns/iter; 2.5655x vs baseline; 1.4335x over previous
import functools

import jax
import jax.numpy as jnp
from jax import lax
from jax.experimental import pallas as pl
from jax.experimental.pallas import tpu as pltpu

N_DEV = 4
M = 4096
K_SHARD = 1024
N_OUT = 2048
HALF = N_OUT // 2
CHUNK = M // N_DEV


def kernel(x, w_mat):
    def body(
        x_hbm,
        w_ref,
        out_hbm,
        xs_ref,
        acc_r_ref,
        acc_l_ref,
        rsr_ref,
        rsl_ref,
        qr_ref,
        ql_ref,
        stage_r_ref,
        stage_l_ref,
        amax_ref,
        copy_sem,
        out_sem_r,
        out_sem_l,
        rs_send_r,
        rs_recv_r,
        rs_send_l,
        rs_recv_l,
        ax_send,
        ax_recv,
        ag_send_r,
        ag_recv_r,
        ag_send_l,
        ag_recv_l,
    ):
        my = lax.axis_index("i")
        right = (my + 1) % N_DEV
        left = (my + N_DEV - 1) % N_DEV

        barrier_sem = pltpu.get_barrier_semaphore()
        for nbr in (left, right):
            pl.semaphore_signal(
                barrier_sem, inc=1,
                device_id=(nbr,), device_id_type=pl.DeviceIdType.MESH,
            )
        pl.semaphore_wait(barrier_sem, 2)

        def load_x(c):
            cp = pltpu.make_async_copy(
                x_hbm.at[pl.ds(c * CHUNK, CHUNK), :], xs_ref, copy_sem
            )
            cp.start()
            cp.wait()

        def dot_half(lo):
            return jnp.dot(
                xs_ref[...],
                w_ref[:, lo : lo + HALF],
                preferred_element_type=jnp.float32,
            )

        def rs_rdma(h):
            rdma_r = pltpu.make_async_remote_copy(
                src_ref=acc_r_ref,
                dst_ref=rsr_ref.at[h],
                send_sem=rs_send_r.at[h],
                recv_sem=rs_recv_r.at[h],
                device_id=(right,),
                device_id_type=pl.DeviceIdType.MESH,
            )
            rdma_l = pltpu.make_async_remote_copy(
                src_ref=acc_l_ref,
                dst_ref=rsl_ref.at[h],
                send_sem=rs_send_l.at[h],
                recv_sem=rs_recv_l.at[h],
                device_id=(left,),
                device_id_type=pl.DeviceIdType.MESH,
            )
            return rdma_r, rdma_l

        load_x(my)
        acc_r_ref[...] = dot_half(0).astype(jnp.bfloat16)
        rdma_r, rdma_l = rs_rdma(0)
        rdma_r.start()
        acc_l_ref[...] = dot_half(HALF).astype(jnp.bfloat16)
        rdma_l.start()
        for h in range(N_DEV - 1):
            load_x((my - h - 1) % N_DEV)
            p_r = dot_half(0)
            if h != 1:
                load_x((my + h + 1) % N_DEV)
            p_l = dot_half(HALF)
            rdma_r.wait()
            acc_r_ref[...] = (rsr_ref[h].astype(jnp.float32) + p_r).astype(
                jnp.bfloat16
            )
            if h + 1 < N_DEV - 1:
                next_r, next_l = rs_rdma(h + 1)
                next_r.start()
            rdma_l.wait()
            acc_l_ref[...] = (rsl_ref[h].astype(jnp.float32) + p_l).astype(
                jnp.bfloat16
            )
            if h + 1 < N_DEV - 1:
                next_l.start()
                rdma_r, rdma_l = next_r, next_l

        own_r = (my + 1) % N_DEV
        own_l = (my + N_DEV - 1) % N_DEV

        my_max = jnp.maximum(
            jnp.max(jnp.abs(acc_r_ref[...].astype(jnp.float32))),
            jnp.max(jnp.abs(acc_l_ref[...].astype(jnp.float32))),
        )
        amax_ref[pl.ds(my, 1)] = jnp.full((1, 8, 128), my_max, jnp.float32)
        ax_rdmas = []
        for k in range(1, N_DEV):
            rdma = pltpu.make_async_remote_copy(
                src_ref=amax_ref.at[pl.ds(my, 1)],
                dst_ref=amax_ref.at[pl.ds(my, 1)],
                send_sem=ax_send.at[k - 1],
                recv_sem=ax_recv.at[k - 1],
                device_id=((my + k) % N_DEV,),
                device_id_type=pl.DeviceIdType.MESH,
            )
            rdma.start()
            ax_rdmas.append(rdma)
        for rdma in ax_rdmas:
            rdma.wait()
        amax = jnp.max(amax_ref[...])
        scale = amax / 127.0

        def quantize(v):
            return jnp.clip(jnp.round(v / scale), -127.0, 127.0).astype(
                jnp.int8
            )

        qr_ref[pl.ds(own_r, 1)] = quantize(acc_r_ref[...].astype(jnp.float32))[
            None
        ]
        ql_ref[pl.ds(own_l, 1)] = quantize(acc_l_ref[...].astype(jnp.float32))[
            None
        ]

        def store_half(q_ref_, stage_ref, sem, c, col_lo):
            stage_ref[...] = q_ref_[pl.ds(c, 1)][0].astype(jnp.float32) * scale
            cp = pltpu.make_async_copy(
                stage_ref,
                out_hbm.at[pl.ds(c * CHUNK, CHUNK), pl.ds(col_lo, HALF)],
                sem,
            )
            cp.start()
            return cp

        for g in range(N_DEV - 1):
            rdma_r = pltpu.make_async_remote_copy(
                src_ref=qr_ref.at[pl.ds((my + 1 - g) % N_DEV, 1)],
                dst_ref=qr_ref.at[pl.ds((my + 1 - g) % N_DEV, 1)],
                send_sem=ag_send_r.at[g],
                recv_sem=ag_recv_r.at[g],
                device_id=(right,),
                device_id_type=pl.DeviceIdType.MESH,
            )
            rdma_l = pltpu.make_async_remote_copy(
                src_ref=ql_ref.at[pl.ds((my - 1 + g) % N_DEV, 1)],
                dst_ref=ql_ref.at[pl.ds((my - 1 + g) % N_DEV, 1)],
                send_sem=ag_send_l.at[g],
                recv_sem=ag_recv_l.at[g],
                device_id=(left,),
                device_id_type=pl.DeviceIdType.MESH,
            )
            rdma_r.start()
            rdma_l.start()
            cp_r = store_half(
                qr_ref, stage_r_ref, out_sem_r, (my + 1 - g) % N_DEV, 0
            )
            cp_l = store_half(
                ql_ref, stage_l_ref, out_sem_l, (my - 1 + g) % N_DEV, HALF
            )
            cp_r.wait()
            cp_l.wait()
            rdma_r.wait()
            rdma_l.wait()
        cp_r = store_half(qr_ref, stage_r_ref, out_sem_r, (my - 2) % N_DEV, 0)
        cp_l = store_half(
            ql_ref, stage_l_ref, out_sem_l, (my + 2) % N_DEV, HALF
        )
        cp_r.wait()
        cp_l.wait()

        @functools.partial(
            pl.run_scoped, second_barrier=pltpu.SemaphoreType.REGULAR
        )
        def _(second_barrier):
            for nbr in (left, right):
                pl.semaphore_signal(
                    second_barrier, inc=1,
                    device_id=(nbr,), device_id_type=pl.DeviceIdType.MESH,
                )
            pl.semaphore_wait(second_barrier, 2)

    return pl.pallas_call(
        body,
        out_shape=jax.ShapeDtypeStruct((M, N_OUT), jnp.float32),
        in_specs=[
            pl.BlockSpec(memory_space=pl.ANY),
            pl.BlockSpec(memory_space=pltpu.VMEM),
        ],
        out_specs=pl.BlockSpec(memory_space=pl.ANY),
        scratch_shapes=[
            pltpu.VMEM((CHUNK, K_SHARD), jnp.float32),
            pltpu.VMEM((CHUNK, HALF), jnp.bfloat16),
            pltpu.VMEM((CHUNK, HALF), jnp.bfloat16),
            pltpu.VMEM((N_DEV - 1, CHUNK, HALF), jnp.bfloat16),
            pltpu.VMEM((N_DEV - 1, CHUNK, HALF), jnp.bfloat16),
            pltpu.VMEM((N_DEV, CHUNK, HALF), jnp.int8),
            pltpu.VMEM((N_DEV, CHUNK, HALF), jnp.int8),
            pltpu.VMEM((CHUNK, HALF), jnp.float32),
            pltpu.VMEM((CHUNK, HALF), jnp.float32),
            pltpu.VMEM((N_DEV, 8, 128), jnp.float32),
            pltpu.SemaphoreType.DMA,
            pltpu.SemaphoreType.DMA,
            pltpu.SemaphoreType.DMA,
            pltpu.SemaphoreType.DMA((N_DEV - 1,)),
            pltpu.SemaphoreType.DMA((N_DEV - 1,)),
            pltpu.SemaphoreType.DMA((N_DEV - 1,)),
            pltpu.SemaphoreType.DMA((N_DEV - 1,)),
            pltpu.SemaphoreType.DMA((N_DEV - 1,)),
            pltpu.SemaphoreType.DMA((N_DEV - 1,)),
            pltpu.SemaphoreType.DMA((N_DEV - 1,)),
            pltpu.SemaphoreType.DMA((N_DEV - 1,)),
            pltpu.SemaphoreType.DMA((N_DEV - 1,)),
            pltpu.SemaphoreType.DMA((N_DEV - 1,)),
        ],
        compiler_params=pltpu.CompilerParams(
            collective_id=0,
            vmem_limit_bytes=128 * 1024 * 1024,
        ),
    )(x, w_mat)
